# baseline (speedup 1.0000x reference)
import functools

import jax
import jax.numpy as jnp
from jax.experimental import pallas as pl
from jax.experimental.pallas import tpu as pltpu

NUM_SAMPLES = 32621
NUM_FEATURES = 2048
BATCH = 128
TEMP = 0.05

BLOCK_N = 512
NUM_BLOCKS = (NUM_SAMPLES + BLOCK_N - 1) // BLOCK_N
PADDED_N = NUM_BLOCKS * BLOCK_N


def _fused_loss_kernel(x_ref, feats_ref, targets_ref, cams_ref, pids_ref,
                       camids_ref, out_ref, xn_ref, acc_ref):
    i = pl.program_id(0)

    @pl.when(i == 0)
    def _init():
        x = x_ref[...]
        norm = jnp.sqrt(jnp.sum(x * x, axis=1, keepdims=True))
        xn_ref[...] = x / jnp.maximum(norm, 1e-12)
        acc_ref[0] = 0.0
        acc_ref[1] = 0.0

    xn = xn_ref[...]
    feats = feats_ref[...]
    out = jax.lax.dot_general(
        xn, feats, (((1,), (1,)), ((), ())),
        preferred_element_type=jnp.float32)
    out = (out + 1.0) * 0.5

    pids = pids_ref[0]
    camids = camids_ref[0]
    pos = targets_ref[...] == pids
    neg = jnp.logical_and(jnp.logical_not(pos), cams_ref[...] == camids)

    alpha_p = jax.nn.relu(1.0 - out)
    alpha_n = jax.nn.relu(out)
    p_terms = jnp.exp(-alpha_p * (out - 1.0) / TEMP)
    n_terms = jnp.exp(alpha_n * out / TEMP)
    acc_ref[0] += jnp.sum(jnp.where(pos, p_terms, 0.0))
    acc_ref[1] += jnp.sum(jnp.where(neg, n_terms, 0.0))

    @pl.when(i == NUM_BLOCKS - 1)
    def _finish():
        out_ref[0, 0] = jnp.log(1.0 + acc_ref[0] * acc_ref[1])


@functools.partial(jax.jit, static_argnames=())
def kernel(inputs, targets, cams, features, pids, camids):
    pad = PADDED_N - NUM_SAMPLES
    pids_p = jnp.pad(pids.astype(jnp.int32), (0, pad), constant_values=-1)
    camids_p = jnp.pad(camids.astype(jnp.int32), (0, pad), constant_values=-1)
    pids_p = pids_p.reshape(NUM_BLOCKS, 1, BLOCK_N)
    camids_p = camids_p.reshape(NUM_BLOCKS, 1, BLOCK_N)
    targets_c = targets.astype(jnp.int32).reshape(BATCH, 1)
    cams_c = cams.astype(jnp.int32).reshape(BATCH, 1)

    res = pl.pallas_call(
        _fused_loss_kernel,
        grid=(NUM_BLOCKS,),
        in_specs=[
            pl.BlockSpec((BATCH, NUM_FEATURES), lambda i: (0, 0)),
            pl.BlockSpec((BLOCK_N, NUM_FEATURES), lambda i: (i, 0)),
            pl.BlockSpec((BATCH, 1), lambda i: (0, 0)),
            pl.BlockSpec((BATCH, 1), lambda i: (0, 0)),
            pl.BlockSpec((1, 1, BLOCK_N), lambda i: (i, 0, 0)),
            pl.BlockSpec((1, 1, BLOCK_N), lambda i: (i, 0, 0)),
        ],
        out_specs=pl.BlockSpec(memory_space=pltpu.SMEM),
        out_shape=jax.ShapeDtypeStruct((1, 1), jnp.float32),
        scratch_shapes=[
            pltpu.VMEM((BATCH, NUM_FEATURES), jnp.float32),
            pltpu.SMEM((2,), jnp.float32),
        ],
        compiler_params=pltpu.CompilerParams(
            dimension_semantics=("arbitrary",)),
    )(inputs, features, targets_c, cams_c, pids_p, camids_p)
    return res[0, 0]


# speedup vs baseline: 1.2267x; 1.2267x over previous
import functools

import jax
import jax.numpy as jnp
from jax.experimental import pallas as pl
from jax.experimental.pallas import tpu as pltpu

NUM_SAMPLES = 32621
NUM_FEATURES = 2048
BATCH = 128
TEMP = 0.05
INV_TEMP = 1.0 / TEMP

BLOCK_N = 1024
NUM_BLOCKS = (NUM_SAMPLES + BLOCK_N - 1) // BLOCK_N
PADDED_N = NUM_BLOCKS * BLOCK_N


def _fused_loss_kernel(x_ref, feats_ref, targets_ref, cams_ref, pids_ref,
                       camids_ref, out_ref, xn_ref, acc_ref):
    i = pl.program_id(0)

    @pl.when(i == 0)
    def _init():
        x = x_ref[...]
        norm = jnp.sqrt(jnp.sum(x * x, axis=1, keepdims=True))
        xn_ref[...] = (x / jnp.maximum(norm, 1e-12)).astype(jnp.bfloat16)
        acc_ref[0] = 0.0
        acc_ref[1] = 0.0

    xn = xn_ref[...]
    feats = feats_ref[...].astype(jnp.bfloat16)
    out = jax.lax.dot_general(
        xn, feats, (((1,), (1,)), ((), ())),
        preferred_element_type=jnp.float32)
    out = (out + 1.0) * 0.5

    pids = pids_ref[0]
    camids = camids_ref[0]
    pos = targets_ref[...] == pids
    neg = jnp.logical_and(jnp.logical_not(pos), cams_ref[...] == camids)

    sel = jnp.where(pos, jax.nn.relu(1.0 - out), jax.nn.relu(out))
    terms = jnp.exp(sel * sel * INV_TEMP)
    acc_ref[0] += jnp.sum(jnp.where(pos, terms, 0.0))
    acc_ref[1] += jnp.sum(jnp.where(neg, terms, 0.0))

    @pl.when(i == NUM_BLOCKS - 1)
    def _finish():
        out_ref[0, 0] = jnp.log(1.0 + acc_ref[0] * acc_ref[1])


@functools.partial(jax.jit, static_argnames=())
def kernel(inputs, targets, cams, features, pids, camids):
    pad = PADDED_N - NUM_SAMPLES
    pids_p = jnp.pad(pids.astype(jnp.int32), (0, pad), constant_values=-1)
    camids_p = jnp.pad(camids.astype(jnp.int32), (0, pad), constant_values=-1)
    pids_p = pids_p.reshape(NUM_BLOCKS, 1, BLOCK_N)
    camids_p = camids_p.reshape(NUM_BLOCKS, 1, BLOCK_N)
    targets_c = targets.astype(jnp.int32).reshape(BATCH, 1)
    cams_c = cams.astype(jnp.int32).reshape(BATCH, 1)

    res = pl.pallas_call(
        _fused_loss_kernel,
        grid=(NUM_BLOCKS,),
        in_specs=[
            pl.BlockSpec((BATCH, NUM_FEATURES), lambda i: (0, 0)),
            pl.BlockSpec((BLOCK_N, NUM_FEATURES), lambda i: (i, 0)),
            pl.BlockSpec((BATCH, 1), lambda i: (0, 0)),
            pl.BlockSpec((BATCH, 1), lambda i: (0, 0)),
            pl.BlockSpec((1, 1, BLOCK_N), lambda i: (i, 0, 0)),
            pl.BlockSpec((1, 1, BLOCK_N), lambda i: (i, 0, 0)),
        ],
        out_specs=pl.BlockSpec(memory_space=pltpu.SMEM),
        out_shape=jax.ShapeDtypeStruct((1, 1), jnp.float32),
        scratch_shapes=[
            pltpu.VMEM((BATCH, NUM_FEATURES), jnp.bfloat16),
            pltpu.SMEM((2,), jnp.float32),
        ],
        compiler_params=pltpu.CompilerParams(
            dimension_semantics=("arbitrary",)),
    )(inputs, features, targets_c, cams_c, pids_p, camids_p)
    return res[0, 0]


# speedup vs baseline: 1.2550x; 1.0231x over previous
import functools

import jax
import jax.numpy as jnp
from jax.experimental import pallas as pl
from jax.experimental.pallas import tpu as pltpu

NUM_SAMPLES = 32621
NUM_FEATURES = 2048
BATCH = 128
TEMP = 0.05
INV_TEMP = 1.0 / TEMP

BLOCK_N = 1024
NUM_BLOCKS = (NUM_SAMPLES + BLOCK_N - 1) // BLOCK_N
PADDED_N = NUM_BLOCKS * BLOCK_N


def _fused_loss_kernel(x_ref, feats_ref, targets_ref, cams_ref, pids_ref,
                       camids_ref, out_ref, xn_ref, acc_ref):
    i = pl.program_id(0)

    @pl.when(i == 0)
    def _init():
        x = x_ref[...]
        norm = jnp.sqrt(jnp.sum(x * x, axis=1, keepdims=True))
        xn_ref[...] = (x / jnp.maximum(norm, 1e-12))
        acc_ref[0] = 0.0
        acc_ref[1] = 0.0

    xn = xn_ref[...]
    feats = feats_ref[...]
    out = jax.lax.dot_general(
        xn, feats, (((1,), (1,)), ((), ())),
        preferred_element_type=jnp.float32,
        precision=jax.lax.Precision.DEFAULT)
    out = (out + 1.0) * 0.5

    pids = pids_ref[0]
    camids = camids_ref[0]
    pos = targets_ref[...] == pids
    neg = jnp.logical_and(jnp.logical_not(pos), cams_ref[...] == camids)

    sel = jnp.where(pos, jax.nn.relu(1.0 - out), jax.nn.relu(out))
    terms = jnp.exp(sel * sel * INV_TEMP)
    acc_ref[0] += jnp.sum(jnp.where(pos, terms, 0.0))
    acc_ref[1] += jnp.sum(jnp.where(neg, terms, 0.0))

    @pl.when(i == NUM_BLOCKS - 1)
    def _finish():
        out_ref[0, 0] = jnp.log(1.0 + acc_ref[0] * acc_ref[1])


@functools.partial(jax.jit, static_argnames=())
def kernel(inputs, targets, cams, features, pids, camids):
    pad = PADDED_N - NUM_SAMPLES
    pids_p = jnp.pad(pids.astype(jnp.int32), (0, pad), constant_values=-1)
    camids_p = jnp.pad(camids.astype(jnp.int32), (0, pad), constant_values=-1)
    pids_p = pids_p.reshape(NUM_BLOCKS, 1, BLOCK_N)
    camids_p = camids_p.reshape(NUM_BLOCKS, 1, BLOCK_N)
    targets_c = targets.astype(jnp.int32).reshape(BATCH, 1)
    cams_c = cams.astype(jnp.int32).reshape(BATCH, 1)

    res = pl.pallas_call(
        _fused_loss_kernel,
        grid=(NUM_BLOCKS,),
        in_specs=[
            pl.BlockSpec((BATCH, NUM_FEATURES), lambda i: (0, 0)),
            pl.BlockSpec((BLOCK_N, NUM_FEATURES), lambda i: (i, 0)),
            pl.BlockSpec((BATCH, 1), lambda i: (0, 0)),
            pl.BlockSpec((BATCH, 1), lambda i: (0, 0)),
            pl.BlockSpec((1, 1, BLOCK_N), lambda i: (i, 0, 0)),
            pl.BlockSpec((1, 1, BLOCK_N), lambda i: (i, 0, 0)),
        ],
        out_specs=pl.BlockSpec(memory_space=pltpu.SMEM),
        out_shape=jax.ShapeDtypeStruct((1, 1), jnp.float32),
        scratch_shapes=[
            pltpu.VMEM((BATCH, NUM_FEATURES), jnp.float32),
            pltpu.SMEM((2,), jnp.float32),
        ],
        compiler_params=pltpu.CompilerParams(
            dimension_semantics=("arbitrary",)),
    )(inputs, features, targets_c, cams_c, pids_p, camids_p)
    return res[0, 0]
